# baseline (speedup 1.0000x reference)
import dataclasses
import functools

import jax
import jax.numpy as jnp
from jax import lax
from jax.experimental import pallas as pl
from jax.experimental.pallas import tpu as pltpu
from jax.experimental.pallas import tpu_sc as plsc

N = 10000
D = 128
E = 320000

NC = 2
NS = 16
NW = NC * NS
L = 16

K = 128
NPAD = 10112
EPT = 10112
CHUNKS = EPT // K
EPAD = EPT * NW
RPT = NPAD // NS

_BLK = 400


def _sc_params():
    cp = pltpu.CompilerParams()
    if "needs_layout_passes" in pltpu.CompilerParams.__dataclass_fields__:
        cp = dataclasses.replace(cp, needs_layout_passes=False)
    return cp


def _prep_body(x_ref, w_ref, m1_ref, m2_ref, g_ref, b_ref, mu_ref, var_ref,
               mapped_ref, s1_ref, s2_ref):
    eps = 1e-3
    scale = g_ref[...] * lax.rsqrt(var_ref[...] + eps)
    xh = (x_ref[...] - mu_ref[...]) * scale + b_ref[...]
    dot = functools.partial(jnp.dot, preferred_element_type=jnp.float32,
                            precision=lax.Precision.HIGHEST)
    mapped_ref[...] = dot(xh, w_ref[...])
    p1 = dot(xh, m1_ref[...])
    s1_ref[...] = jnp.tanh(jnp.sum(p1 * xh, axis=1, keepdims=True))
    p2 = dot(xh, m2_ref[...])
    s2_ref[...] = jnp.tanh(jnp.sum(p2 * xh, axis=1, keepdims=True))


def _prep(x, W, M1, M2, gamma, beta, mu, var):
    full = pl.BlockSpec((D, D), lambda i: (0, 0))
    vec = pl.BlockSpec((1, D), lambda i: (0, 0))
    return pl.pallas_call(
        _prep_body,
        grid=(N // _BLK,),
        in_specs=[pl.BlockSpec((_BLK, D), lambda i: (i, 0)),
                  full, full, full, vec, vec, vec, vec],
        out_specs=[pl.BlockSpec((_BLK, D), lambda i: (i, 0)),
                   pl.BlockSpec((_BLK, 1), lambda i: (i, 0)),
                   pl.BlockSpec((_BLK, 1), lambda i: (i, 0))],
        out_shape=[jax.ShapeDtypeStruct((N, D), jnp.float32),
                   jax.ShapeDtypeStruct((N, 1), jnp.float32),
                   jax.ShapeDtypeStruct((N, 1), jnp.float32)],
    )(x, W, M1, M2, gamma.reshape(1, D), beta.reshape(1, D),
      mu.reshape(1, D), var.reshape(1, D))


def _exw_body(s1_hbm, s2_hbm, row_hbm, col_hbm, a_hbm, ex_hbm, den_hbm,
              s1_v, s2_v, row_v, col_v, a_v, ex_v, zden, den_sp, sem_d):
    cid = lax.axis_index("c")
    sid = lax.axis_index("s")
    wid = cid * NS + sid

    pltpu.sync_copy(s1_hbm, s1_v)
    pltpu.sync_copy(s2_hbm, s2_v)
    pltpu.sync_copy(row_hbm.at[wid], row_v)
    pltpu.sync_copy(col_hbm.at[wid], col_v)
    pltpu.sync_copy(a_hbm.at[wid], a_v)

    zero16 = jnp.zeros((L,), jnp.float32)
    for t in range(640 // L):
        zden[pl.ds(t * L, L)] = zero16
    rbase = sid * RPT
    pltpu.sync_copy(zden.at[pl.ds(0, RPT)], den_sp.at[pl.ds(rbase, RPT)])

    @pl.loop(0, CHUNKS)
    def _chunk(cc):
        for j in range(K // L):
            r16 = row_v[cc, pl.ds(j * L, L)]
            c16 = col_v[cc, pl.ds(j * L, L)]
            av = a_v[cc, pl.ds(j * L, L)]
            g1 = plsc.load_gather(s1_v, [r16])
            g2 = plsc.load_gather(s2_v, [c16])
            ev = av * g1 + av * g2
            ev = jnp.where(ev >= 0.0, ev, 0.2 * ev)
            ex_v[cc, pl.ds(j * L, L)] = jnp.exp(ev)

    pltpu.sync_copy(ex_v, ex_hbm.at[wid])

    plsc.subcore_barrier()

    @pl.loop(0, CHUNKS)
    def _fire(cc):
        pltpu.async_copy(ex_v.at[cc], den_sp.at[row_v.at[cc]], sem_d,
                         add=True)

    @pl.loop(0, CHUNKS)
    def _drain(cc):
        pltpu.make_async_copy(ex_v.at[0], den_sp.at[pl.ds(0, K)],
                              sem_d).wait()

    plsc.subcore_barrier()

    pltpu.sync_copy(den_sp.at[pl.ds(rbase, RPT)], zden.at[pl.ds(0, RPT)])
    pltpu.sync_copy(zden.at[pl.ds(0, RPT)],
                    den_hbm.at[pl.ds(cid * NPAD + rbase, RPT)])


def _exw(s1p, s2p, rowc, colc, ac):
    mesh = plsc.VectorSubcoreMesh(core_axis_name="c", subcore_axis_name="s")
    f = pl.kernel(
        _exw_body,
        out_type=(jax.ShapeDtypeStruct((NW, CHUNKS, K), jnp.float32),
                  jax.ShapeDtypeStruct((NC * NPAD,), jnp.float32)),
        mesh=mesh,
        scratch_types=[
            pltpu.VMEM((NPAD,), jnp.float32),
            pltpu.VMEM((NPAD,), jnp.float32),
            pltpu.VMEM((CHUNKS, K), jnp.int32),
            pltpu.VMEM((CHUNKS, K), jnp.int32),
            pltpu.VMEM((CHUNKS, K), jnp.float32),
            pltpu.VMEM((CHUNKS, K), jnp.float32),
            pltpu.VMEM((640,), jnp.float32),
            pltpu.VMEM_SHARED((NPAD,), jnp.float32),
            pltpu.SemaphoreType.DMA,
        ],
        compiler_params=_sc_params(),
    )
    return f(s1p, s2p, rowc, colc, ac)


def _edge_body(mapped_hbm, row_hbm, col_hbm, ex_hbm,
               acc_hbm,
               row_r, col_r, ex_r, gbuf_a, gbuf_b, zden,
               acc_sp, sem_g, sem_s, sem_row, sem_col, sem_ex):
    cid = lax.axis_index("c")
    sid = lax.axis_index("s")
    wid = cid * NS + sid
    cbase = wid * CHUNKS

    zero16 = jnp.zeros((L,), jnp.float32)

    @pl.loop(0, K)
    def _zg(i):
        for k in range(D // L):
            gbuf_a[i, pl.ds(k * L, L)] = zero16

    for t in range(640 // L):
        zden[pl.ds(t * L, L)] = zero16

    rbase = sid * RPT
    for t in range(RPT // K):
        pltpu.sync_copy(gbuf_a, acc_sp.at[pl.ds(rbase + t * K, K)])
    rem = RPT - (RPT // K) * K
    if rem:
        pltpu.sync_copy(gbuf_a.at[pl.ds(0, rem)],
                        acc_sp.at[pl.ds(rbase + (RPT // K) * K, rem)])

    plsc.subcore_barrier()

    def idx_start(cc):
        s = lax.rem(cc, 3)
        pltpu.async_copy(row_hbm.at[cbase + cc], row_r.at[s], sem_row)
        pltpu.async_copy(col_hbm.at[cbase + cc], col_r.at[s], sem_col)
        pltpu.async_copy(ex_hbm.at[cbase + cc], ex_r.at[s], sem_ex)

    def idx_wait():
        pltpu.make_async_copy(row_hbm.at[0], row_r.at[0], sem_row).wait()
        pltpu.make_async_copy(col_hbm.at[0], col_r.at[0], sem_col).wait()
        pltpu.make_async_copy(ex_hbm.at[0], ex_r.at[0], sem_ex).wait()

    def gather_start(cc, buf):
        s = lax.rem(cc, 3)
        pltpu.async_copy(mapped_hbm.at[col_r.at[s, 0]], buf, sem_g)

    def gather_wait(buf):
        pltpu.make_async_copy(mapped_hbm.at[pl.ds(0, K)], buf, sem_g).wait()

    def scatter_wait():
        pltpu.make_async_copy(gbuf_a, acc_sp.at[pl.ds(0, K)], sem_s).wait()

    def process(c, buf, other_buf, first_guard, idx_guard):
        s = lax.rem(c, 3)
        idx_wait()
        gather_wait(buf)
        if first_guard:
            @pl.when(c >= 1)
            def _s():
                scatter_wait()
        else:
            scatter_wait()
        gather_start(c + 1, other_buf)
        if idx_guard:
            @pl.when(c + 2 < CHUNKS)
            def _i():
                idx_start(c + 2)
        else:
            idx_start(c + 2)

        for grp in range(K // L):
            base = grp * L
            exg = ex_r[s, 0, pl.ds(base, L)]
            for l in range(L):
                i = base + l
                e = exg[l]
                for k in range(D // L):
                    buf[i, pl.ds(k * L, L)] = buf[i, pl.ds(k * L, L)] * e

        pltpu.async_copy(buf, acc_sp.at[row_r.at[s, 0]], sem_s, add=True)

    idx_start(0)
    idx_start(1)
    idx_wait()
    gather_start(0, gbuf_a)

    @pl.loop(0, (CHUNKS - 1) // 2)
    def _pipe(p):
        a = pl.multiple_of(2 * p, 2)
        process(a, gbuf_a, gbuf_b, True, False)
        process(a + 1, gbuf_b, gbuf_a, False, True)

    cl = CHUNKS - 1
    sl = cl % 3
    gather_wait(gbuf_a)
    scatter_wait()
    for grp in range(K // L):
        base = grp * L
        exg = ex_r[sl, 0, pl.ds(base, L)]
        for l in range(L):
            e = exg[l]
            for k in range(D // L):
                gbuf_a[base + l, pl.ds(k * L, L)] = (
                    gbuf_a[base + l, pl.ds(k * L, L)] * e)
    pltpu.async_copy(gbuf_a, acc_sp.at[row_r.at[sl, 0]], sem_s, add=True)
    scatter_wait()

    plsc.subcore_barrier()

    pltpu.sync_copy(acc_sp.at[pl.ds(rbase, RPT)],
                    acc_hbm.at[cid, pl.ds(rbase, RPT)])


def _edge(mapped, rowc, colc, exc):
    mesh = plsc.VectorSubcoreMesh(core_axis_name="c", subcore_axis_name="s")
    f = pl.kernel(
        _edge_body,
        out_type=jax.ShapeDtypeStruct((NC, NPAD, D), jnp.float32),
        mesh=mesh,
        scratch_types=[
            pltpu.VMEM((3, 1, K), jnp.int32),
            pltpu.VMEM((3, 1, K), jnp.int32),
            pltpu.VMEM((3, 1, K), jnp.float32),
            pltpu.VMEM((K, D), jnp.float32),
            pltpu.VMEM((K, D), jnp.float32),
            pltpu.VMEM((640,), jnp.float32),
            pltpu.VMEM_SHARED((NPAD, D), jnp.float32),
            pltpu.SemaphoreType.DMA,
            pltpu.SemaphoreType.DMA,
            pltpu.SemaphoreType.DMA,
            pltpu.SemaphoreType.DMA,
            pltpu.SemaphoreType.DMA,
        ],
        compiler_params=_sc_params(),
    )
    return f(mapped, rowc, colc, exc)


def _final_body(acc_ref, den_ref, out_ref):
    acc = acc_ref[0] + acc_ref[1]
    den = den_ref[0, :, :] + den_ref[1, :, :]
    out_ref[...] = acc * (1.0 / jnp.maximum(den, 1e-30))


def _final(acc, den):
    return pl.pallas_call(
        _final_body,
        grid=(N // _BLK,),
        in_specs=[pl.BlockSpec((NC, _BLK, D), lambda i: (0, i, 0)),
                  pl.BlockSpec((NC, _BLK, 1), lambda i: (0, i, 0))],
        out_specs=pl.BlockSpec((_BLK, D), lambda i: (i, 0)),
        out_shape=jax.ShapeDtypeStruct((N, D), jnp.float32),
    )(acc, den)


def kernel(x, edge_index, a_vals, W, M1, M2, gamma, beta, moving_mean,
           moving_var):
    mapped, s1, s2 = _prep(x, W, M1, M2, gamma, beta, moving_mean, moving_var)
    row = edge_index[0]
    col = edge_index[1]
    pad = EPAD - E
    zpad = jnp.zeros((NPAD - N,), jnp.float32)
    s1p = jnp.concatenate([s1.reshape(N), zpad])
    s2p = jnp.concatenate([s2.reshape(N), zpad])
    rowp = jnp.concatenate([row, jnp.full((pad,), N, jnp.int32)])
    colp = jnp.concatenate([col, jnp.zeros((pad,), jnp.int32)])
    ap = jnp.concatenate([a_vals, jnp.zeros((pad,), jnp.float32)])
    ex, den = _exw(s1p, s2p, rowp.reshape(NW, CHUNKS, K),
                   colp.reshape(NW, CHUNKS, K), ap.reshape(NW, CHUNKS, K))
    shape3 = (NW * CHUNKS, 1, K)
    acc = _edge(mapped, rowp.reshape(shape3), colp.reshape(shape3),
                ex.reshape(shape3))
    return _final(acc, den.reshape(NC, NPAD, 1))


# speedup vs baseline: 1.0921x; 1.0921x over previous
import dataclasses
import functools

import jax
import jax.numpy as jnp
from jax import lax
from jax.experimental import pallas as pl
from jax.experimental.pallas import tpu as pltpu
from jax.experimental.pallas import tpu_sc as plsc

N = 10000
D = 128
E = 320000

NC = 2
NS = 16
NW = NC * NS
L = 16

K = 128
NPAD = 10112
EPT = 10112
CHUNKS = EPT // K
EPAD = EPT * NW
RPT = NPAD // NS

_BLK = 400


def _sc_params():
    cp = pltpu.CompilerParams()
    if "needs_layout_passes" in pltpu.CompilerParams.__dataclass_fields__:
        cp = dataclasses.replace(cp, needs_layout_passes=False)
    return cp


def _prep_body(x_ref, w_ref, m1_ref, m2_ref, g_ref, b_ref, mu_ref, var_ref,
               mapped_ref, s1_ref, s2_ref):
    eps = 1e-3
    scale = g_ref[...] * lax.rsqrt(var_ref[...] + eps)
    xh = (x_ref[...] - mu_ref[...]) * scale + b_ref[...]
    dot = functools.partial(jnp.dot, preferred_element_type=jnp.float32,
                            precision=lax.Precision.HIGHEST)
    mapped_ref[...] = dot(xh, w_ref[...])
    p1 = dot(xh, m1_ref[...])
    s1_ref[...] = jnp.tanh(jnp.sum(p1 * xh, axis=1, keepdims=True))
    p2 = dot(xh, m2_ref[...])
    s2_ref[...] = jnp.tanh(jnp.sum(p2 * xh, axis=1, keepdims=True))


def _prep(x, W, M1, M2, gamma, beta, mu, var):
    full = pl.BlockSpec((D, D), lambda i: (0, 0))
    vec = pl.BlockSpec((1, D), lambda i: (0, 0))
    return pl.pallas_call(
        _prep_body,
        grid=(N // _BLK,),
        in_specs=[pl.BlockSpec((_BLK, D), lambda i: (i, 0)),
                  full, full, full, vec, vec, vec, vec],
        out_specs=[pl.BlockSpec((_BLK, D), lambda i: (i, 0)),
                   pl.BlockSpec((_BLK, 1), lambda i: (i, 0)),
                   pl.BlockSpec((_BLK, 1), lambda i: (i, 0))],
        out_shape=[jax.ShapeDtypeStruct((N, D), jnp.float32),
                   jax.ShapeDtypeStruct((N, 1), jnp.float32),
                   jax.ShapeDtypeStruct((N, 1), jnp.float32)],
    )(x, W, M1, M2, gamma.reshape(1, D), beta.reshape(1, D),
      mu.reshape(1, D), var.reshape(1, D))


def _exw_body(s1_hbm, s2_hbm, row_hbm, col_hbm, a_hbm, ex_hbm, den_hbm,
              s1_v, s2_v, row_v, col_v, a_v, ex_v, zden, den_sp, sem_d):
    cid = lax.axis_index("c")
    sid = lax.axis_index("s")
    wid = cid * NS + sid

    pltpu.sync_copy(s1_hbm, s1_v)
    pltpu.sync_copy(s2_hbm, s2_v)
    pltpu.sync_copy(row_hbm.at[wid], row_v)
    pltpu.sync_copy(col_hbm.at[wid], col_v)
    pltpu.sync_copy(a_hbm.at[wid], a_v)

    zero16 = jnp.zeros((L,), jnp.float32)
    for t in range(640 // L):
        zden[pl.ds(t * L, L)] = zero16
    rbase = sid * RPT
    pltpu.sync_copy(zden.at[pl.ds(0, RPT)], den_sp.at[pl.ds(rbase, RPT)])

    @pl.loop(0, CHUNKS)
    def _chunk(cc):
        for j in range(K // L):
            r16 = row_v[cc, pl.ds(j * L, L)]
            c16 = col_v[cc, pl.ds(j * L, L)]
            av = a_v[cc, pl.ds(j * L, L)]
            g1 = plsc.load_gather(s1_v, [r16])
            g2 = plsc.load_gather(s2_v, [c16])
            ev = av * g1 + av * g2
            ev = jnp.where(ev >= 0.0, ev, 0.2 * ev)
            ex_v[cc, pl.ds(j * L, L)] = jnp.exp(ev)

    pltpu.sync_copy(ex_v, ex_hbm.at[wid])

    plsc.subcore_barrier()

    @pl.loop(0, CHUNKS)
    def _fire(cc):
        pltpu.async_copy(ex_v.at[cc], den_sp.at[row_v.at[cc]], sem_d,
                         add=True)

    @pl.loop(0, CHUNKS)
    def _drain(cc):
        pltpu.make_async_copy(ex_v.at[0], den_sp.at[pl.ds(0, K)],
                              sem_d).wait()

    plsc.subcore_barrier()

    pltpu.sync_copy(den_sp.at[pl.ds(rbase, RPT)], zden.at[pl.ds(0, RPT)])
    pltpu.sync_copy(zden.at[pl.ds(0, RPT)],
                    den_hbm.at[pl.ds(cid * NPAD + rbase, RPT)])


def _exw(s1p, s2p, rowc, colc, ac):
    mesh = plsc.VectorSubcoreMesh(core_axis_name="c", subcore_axis_name="s")
    f = pl.kernel(
        _exw_body,
        out_type=(jax.ShapeDtypeStruct((NW, CHUNKS, K), jnp.float32),
                  jax.ShapeDtypeStruct((NC * NPAD,), jnp.float32)),
        mesh=mesh,
        scratch_types=[
            pltpu.VMEM((NPAD,), jnp.float32),
            pltpu.VMEM((NPAD,), jnp.float32),
            pltpu.VMEM((CHUNKS, K), jnp.int32),
            pltpu.VMEM((CHUNKS, K), jnp.int32),
            pltpu.VMEM((CHUNKS, K), jnp.float32),
            pltpu.VMEM((CHUNKS, K), jnp.float32),
            pltpu.VMEM((640,), jnp.float32),
            pltpu.VMEM_SHARED((NPAD,), jnp.float32),
            pltpu.SemaphoreType.DMA,
        ],
        compiler_params=_sc_params(),
    )
    return f(s1p, s2p, rowc, colc, ac)


def _edge_body(mapped_hbm, row_hbm, col_hbm, ex_hbm,
               acc_hbm,
               row_r, col_r, ex_r, gbuf_a, gbuf_b, zden,
               acc_sp, sem_g, sem_s, sem_row, sem_col, sem_ex):
    cid = lax.axis_index("c")
    sid = lax.axis_index("s")
    wid = cid * NS + sid
    cbase = wid * CHUNKS

    zero16 = jnp.zeros((L,), jnp.float32)

    @pl.loop(0, K)
    def _zg(i):
        for k in range(D // L):
            gbuf_a[i, pl.ds(k * L, L)] = zero16

    for t in range(640 // L):
        zden[pl.ds(t * L, L)] = zero16

    rbase = sid * RPT
    for t in range(RPT // K):
        pltpu.sync_copy(gbuf_a, acc_sp.at[pl.ds(rbase + t * K, K)])
    rem = RPT - (RPT // K) * K
    if rem:
        pltpu.sync_copy(gbuf_a.at[pl.ds(0, rem)],
                        acc_sp.at[pl.ds(rbase + (RPT // K) * K, rem)])

    plsc.subcore_barrier()

    def idx_start(cc):
        s = lax.rem(cc, 3)
        pltpu.async_copy(row_hbm.at[cbase + cc], row_r.at[s], sem_row)
        pltpu.async_copy(col_hbm.at[cbase + cc], col_r.at[s], sem_col)
        pltpu.async_copy(ex_hbm.at[cbase + cc], ex_r.at[s], sem_ex)

    def idx_wait():
        pltpu.make_async_copy(row_hbm.at[0], row_r.at[0], sem_row).wait()
        pltpu.make_async_copy(col_hbm.at[0], col_r.at[0], sem_col).wait()
        pltpu.make_async_copy(ex_hbm.at[0], ex_r.at[0], sem_ex).wait()

    def gather_start(cc, buf):
        s = lax.rem(cc, 3)
        pltpu.async_copy(mapped_hbm.at[col_r.at[s, 0]], buf, sem_g)

    def gather_wait(buf):
        pltpu.make_async_copy(mapped_hbm.at[pl.ds(0, K)], buf, sem_g).wait()

    def scatter_wait():
        pltpu.make_async_copy(gbuf_a, acc_sp.at[pl.ds(0, K)], sem_s).wait()

    def process(c, buf, other_buf, first_guard, idx_guard):
        s = lax.rem(c, 3)
        idx_wait()
        gather_wait(buf)
        if first_guard:
            @pl.when(c >= 1)
            def _s():
                scatter_wait()
        else:
            scatter_wait()
        gather_start(c + 1, other_buf)
        if idx_guard:
            @pl.when(c + 2 < CHUNKS)
            def _i():
                idx_start(c + 2)
        else:
            idx_start(c + 2)

        @pl.loop(0, K // L)
        def _scale(grp):
            base = pl.multiple_of(grp * L, L)
            exg = ex_r[s, 0, pl.ds(base, L)]
            for l in range(L):
                i = base + l
                e = exg[l]
                for k in range(D // L):
                    buf[i, pl.ds(k * L, L)] = buf[i, pl.ds(k * L, L)] * e

        pltpu.async_copy(buf, acc_sp.at[row_r.at[s, 0]], sem_s, add=True)

    idx_start(0)
    idx_start(1)
    idx_wait()
    gather_start(0, gbuf_a)

    @pl.loop(0, (CHUNKS - 1) // 2)
    def _pipe(p):
        a = pl.multiple_of(2 * p, 2)
        process(a, gbuf_a, gbuf_b, True, False)
        process(a + 1, gbuf_b, gbuf_a, False, True)

    cl = CHUNKS - 1
    sl = cl % 3
    gather_wait(gbuf_a)
    scatter_wait()

    @pl.loop(0, K // L)
    def _scale_last(grp):
        base = pl.multiple_of(grp * L, L)
        exg = ex_r[sl, 0, pl.ds(base, L)]
        for l in range(L):
            e = exg[l]
            for k in range(D // L):
                gbuf_a[base + l, pl.ds(k * L, L)] = (
                    gbuf_a[base + l, pl.ds(k * L, L)] * e)

    pltpu.async_copy(gbuf_a, acc_sp.at[row_r.at[sl, 0]], sem_s, add=True)
    scatter_wait()

    plsc.subcore_barrier()

    pltpu.sync_copy(acc_sp.at[pl.ds(rbase, RPT)],
                    acc_hbm.at[cid, pl.ds(rbase, RPT)])


def _edge(mapped, rowc, colc, exc):
    mesh = plsc.VectorSubcoreMesh(core_axis_name="c", subcore_axis_name="s")
    f = pl.kernel(
        _edge_body,
        out_type=jax.ShapeDtypeStruct((NC, NPAD, D), jnp.float32),
        mesh=mesh,
        scratch_types=[
            pltpu.VMEM((3, 1, K), jnp.int32),
            pltpu.VMEM((3, 1, K), jnp.int32),
            pltpu.VMEM((3, 1, K), jnp.float32),
            pltpu.VMEM((K, D), jnp.float32),
            pltpu.VMEM((K, D), jnp.float32),
            pltpu.VMEM((640,), jnp.float32),
            pltpu.VMEM_SHARED((NPAD, D), jnp.float32),
            pltpu.SemaphoreType.DMA,
            pltpu.SemaphoreType.DMA,
            pltpu.SemaphoreType.DMA,
            pltpu.SemaphoreType.DMA,
            pltpu.SemaphoreType.DMA,
        ],
        compiler_params=_sc_params(),
    )
    return f(mapped, rowc, colc, exc)


def _final_body(acc_ref, den_ref, out_ref):
    acc = acc_ref[0] + acc_ref[1]
    den = den_ref[0, :, :] + den_ref[1, :, :]
    out_ref[...] = acc * (1.0 / jnp.maximum(den, 1e-30))


def _final(acc, den):
    return pl.pallas_call(
        _final_body,
        grid=(N // _BLK,),
        in_specs=[pl.BlockSpec((NC, _BLK, D), lambda i: (0, i, 0)),
                  pl.BlockSpec((NC, _BLK, 1), lambda i: (0, i, 0))],
        out_specs=pl.BlockSpec((_BLK, D), lambda i: (i, 0)),
        out_shape=jax.ShapeDtypeStruct((N, D), jnp.float32),
    )(acc, den)


def kernel(x, edge_index, a_vals, W, M1, M2, gamma, beta, moving_mean,
           moving_var):
    mapped, s1, s2 = _prep(x, W, M1, M2, gamma, beta, moving_mean, moving_var)
    row = edge_index[0]
    col = edge_index[1]
    pad = EPAD - E
    zpad = jnp.zeros((NPAD - N,), jnp.float32)
    s1p = jnp.concatenate([s1.reshape(N), zpad])
    s2p = jnp.concatenate([s2.reshape(N), zpad])
    rowp = jnp.concatenate([row, jnp.full((pad,), N, jnp.int32)])
    colp = jnp.concatenate([col, jnp.zeros((pad,), jnp.int32)])
    ap = jnp.concatenate([a_vals, jnp.zeros((pad,), jnp.float32)])
    ex, den = _exw(s1p, s2p, rowp.reshape(NW, CHUNKS, K),
                   colp.reshape(NW, CHUNKS, K), ap.reshape(NW, CHUNKS, K))
    shape3 = (NW * CHUNKS, 1, K)
    acc = _edge(mapped, rowp.reshape(shape3), colp.reshape(shape3),
                ex.reshape(shape3))
    return _final(acc, den.reshape(NC, NPAD, 1))


# speedup vs baseline: 1.1787x; 1.0793x over previous
import dataclasses
import functools

import jax
import jax.numpy as jnp
from jax import lax
from jax.experimental import pallas as pl
from jax.experimental.pallas import tpu as pltpu
from jax.experimental.pallas import tpu_sc as plsc

N = 10000
D = 128
E = 320000

NC = 2
NS = 16
NW = NC * NS
L = 16

K = 128
NPAD = 10112
EPT = 10112
CHUNKS = EPT // K
EPAD = EPT * NW
CH0 = 105
CH1 = 53
RPT = NPAD // NS

_BLK = 400


def _sc_params():
    cp = pltpu.CompilerParams()
    if "needs_layout_passes" in pltpu.CompilerParams.__dataclass_fields__:
        cp = dataclasses.replace(cp, needs_layout_passes=False)
    return cp


def _prep_body(x_ref, w_ref, m1_ref, m2_ref, g_ref, b_ref, mu_ref, var_ref,
               mapped_ref, s1_ref, s2_ref):
    eps = 1e-3
    scale = g_ref[...] * lax.rsqrt(var_ref[...] + eps)
    xh = (x_ref[...] - mu_ref[...]) * scale + b_ref[...]
    dot = functools.partial(jnp.dot, preferred_element_type=jnp.float32,
                            precision=lax.Precision.HIGHEST)
    mapped_ref[...] = dot(xh, w_ref[...])
    p1 = dot(xh, m1_ref[...])
    s1_ref[...] = jnp.tanh(jnp.sum(p1 * xh, axis=1, keepdims=True))
    p2 = dot(xh, m2_ref[...])
    s2_ref[...] = jnp.tanh(jnp.sum(p2 * xh, axis=1, keepdims=True))


def _prep(x, W, M1, M2, gamma, beta, mu, var):
    full = pl.BlockSpec((D, D), lambda i: (0, 0))
    vec = pl.BlockSpec((1, D), lambda i: (0, 0))
    return pl.pallas_call(
        _prep_body,
        grid=(N // _BLK,),
        in_specs=[pl.BlockSpec((_BLK, D), lambda i: (i, 0)),
                  full, full, full, vec, vec, vec, vec],
        out_specs=[pl.BlockSpec((_BLK, D), lambda i: (i, 0)),
                   pl.BlockSpec((_BLK, 1), lambda i: (i, 0)),
                   pl.BlockSpec((_BLK, 1), lambda i: (i, 0))],
        out_shape=[jax.ShapeDtypeStruct((N, D), jnp.float32),
                   jax.ShapeDtypeStruct((N, 1), jnp.float32),
                   jax.ShapeDtypeStruct((N, 1), jnp.float32)],
    )(x, W, M1, M2, gamma.reshape(1, D), beta.reshape(1, D),
      mu.reshape(1, D), var.reshape(1, D))


def _exw_body(s1_hbm, s2_hbm, row_hbm, col_hbm, a_hbm, ex_hbm, den_hbm,
              s1_v, s2_v, row_v, col_v, a_v, ex_v, zden, den_sp, sem_d):
    cid = lax.axis_index("c")
    sid = lax.axis_index("s")
    wid = cid * NS + sid

    pltpu.sync_copy(s1_hbm, s1_v)
    pltpu.sync_copy(s2_hbm, s2_v)
    pltpu.sync_copy(row_hbm.at[wid], row_v)
    pltpu.sync_copy(col_hbm.at[wid], col_v)
    pltpu.sync_copy(a_hbm.at[wid], a_v)

    zero16 = jnp.zeros((L,), jnp.float32)
    for t in range(640 // L):
        zden[pl.ds(t * L, L)] = zero16
    rbase = sid * RPT
    pltpu.sync_copy(zden.at[pl.ds(0, RPT)], den_sp.at[pl.ds(rbase, RPT)])

    @pl.loop(0, CHUNKS)
    def _chunk(cc):
        for j in range(K // L):
            r16 = row_v[cc, pl.ds(j * L, L)]
            c16 = col_v[cc, pl.ds(j * L, L)]
            av = a_v[cc, pl.ds(j * L, L)]
            g1 = plsc.load_gather(s1_v, [r16])
            g2 = plsc.load_gather(s2_v, [c16])
            ev = av * g1 + av * g2
            ev = jnp.where(ev >= 0.0, ev, 0.2 * ev)
            ex_v[cc, pl.ds(j * L, L)] = jnp.exp(ev)

    pltpu.sync_copy(ex_v, ex_hbm.at[wid])

    plsc.subcore_barrier()

    @pl.loop(0, CHUNKS)
    def _fire(cc):
        pltpu.async_copy(ex_v.at[cc], den_sp.at[row_v.at[cc]], sem_d,
                         add=True)

    @pl.loop(0, CHUNKS)
    def _drain(cc):
        pltpu.make_async_copy(ex_v.at[0], den_sp.at[pl.ds(0, K)],
                              sem_d).wait()

    plsc.subcore_barrier()

    pltpu.sync_copy(den_sp.at[pl.ds(rbase, RPT)], zden.at[pl.ds(0, RPT)])
    pltpu.sync_copy(zden.at[pl.ds(0, RPT)],
                    den_hbm.at[pl.ds(cid * NPAD + rbase, RPT)])


def _exw(s1p, s2p, rowc, colc, ac):
    mesh = plsc.VectorSubcoreMesh(core_axis_name="c", subcore_axis_name="s")
    f = pl.kernel(
        _exw_body,
        out_type=(jax.ShapeDtypeStruct((NW, CHUNKS, K), jnp.float32),
                  jax.ShapeDtypeStruct((NC * NPAD,), jnp.float32)),
        mesh=mesh,
        scratch_types=[
            pltpu.VMEM((NPAD,), jnp.float32),
            pltpu.VMEM((NPAD,), jnp.float32),
            pltpu.VMEM((CHUNKS, K), jnp.int32),
            pltpu.VMEM((CHUNKS, K), jnp.int32),
            pltpu.VMEM((CHUNKS, K), jnp.float32),
            pltpu.VMEM((CHUNKS, K), jnp.float32),
            pltpu.VMEM((640,), jnp.float32),
            pltpu.VMEM_SHARED((NPAD,), jnp.float32),
            pltpu.SemaphoreType.DMA,
        ],
        compiler_params=_sc_params(),
    )
    return f(s1p, s2p, rowc, colc, ac)


def _edge_body(mapped_hbm, row_hbm, col_hbm, ex_hbm,
               acc_hbm,
               row_r, col_r, ex_r, gbuf_a, gbuf_b, zden,
               acc_sp, sem_g, sem_s, sem_row, sem_col, sem_ex):
    cid = lax.axis_index("c")
    sid = lax.axis_index("s")
    nch = jnp.where(cid == 0, CH0, CH1)
    cbase = jnp.where(cid == 0, sid * CH0, NS * CH0 + sid * CH1)

    zero16 = jnp.zeros((L,), jnp.float32)

    @pl.loop(0, K)
    def _zg(i):
        for k in range(D // L):
            gbuf_a[i, pl.ds(k * L, L)] = zero16

    for t in range(640 // L):
        zden[pl.ds(t * L, L)] = zero16

    rbase = sid * RPT
    for t in range(RPT // K):
        pltpu.sync_copy(gbuf_a, acc_sp.at[pl.ds(rbase + t * K, K)])
    rem = RPT - (RPT // K) * K
    if rem:
        pltpu.sync_copy(gbuf_a.at[pl.ds(0, rem)],
                        acc_sp.at[pl.ds(rbase + (RPT // K) * K, rem)])

    plsc.subcore_barrier()

    def idx_start(cc):
        s = lax.rem(cc, 3)
        pltpu.async_copy(row_hbm.at[cbase + cc], row_r.at[s], sem_row)
        pltpu.async_copy(col_hbm.at[cbase + cc], col_r.at[s], sem_col)
        pltpu.async_copy(ex_hbm.at[cbase + cc], ex_r.at[s], sem_ex)

    def idx_wait():
        pltpu.make_async_copy(row_hbm.at[0], row_r.at[0], sem_row).wait()
        pltpu.make_async_copy(col_hbm.at[0], col_r.at[0], sem_col).wait()
        pltpu.make_async_copy(ex_hbm.at[0], ex_r.at[0], sem_ex).wait()

    def gather_start(cc, buf):
        s = lax.rem(cc, 3)
        pltpu.async_copy(mapped_hbm.at[col_r.at[s, 0]], buf, sem_g)

    def gather_wait(buf):
        pltpu.make_async_copy(mapped_hbm.at[pl.ds(0, K)], buf, sem_g).wait()

    def scatter_wait():
        pltpu.make_async_copy(gbuf_a, acc_sp.at[pl.ds(0, K)], sem_s).wait()

    def process(c, buf, other_buf, first_guard, idx_guard):
        s = lax.rem(c, 3)
        idx_wait()
        gather_wait(buf)
        if first_guard:
            @pl.when(c >= 1)
            def _s():
                scatter_wait()
        else:
            scatter_wait()
        gather_start(c + 1, other_buf)
        if idx_guard:
            @pl.when(c + 2 < nch)
            def _i():
                idx_start(c + 2)
        else:
            idx_start(c + 2)

        @pl.loop(0, K // L)
        def _scale(grp):
            base = pl.multiple_of(grp * L, L)
            exg = ex_r[s, 0, pl.ds(base, L)]
            for l in range(L):
                i = base + l
                e = exg[l]
                for k in range(D // L):
                    buf[i, pl.ds(k * L, L)] = buf[i, pl.ds(k * L, L)] * e

        pltpu.async_copy(buf, acc_sp.at[row_r.at[s, 0]], sem_s, add=True)

    idx_start(0)
    idx_start(1)
    idx_wait()
    gather_start(0, gbuf_a)

    @pl.loop(0, (nch - 1) // 2)
    def _pipe(p):
        a = pl.multiple_of(2 * p, 2)
        process(a, gbuf_a, gbuf_b, True, False)
        process(a + 1, gbuf_b, gbuf_a, False, True)

    cl = nch - 1
    sl = lax.rem(cl, 3)
    gather_wait(gbuf_a)
    scatter_wait()

    @pl.loop(0, K // L)
    def _scale_last(grp):
        base = pl.multiple_of(grp * L, L)
        exg = ex_r[sl, 0, pl.ds(base, L)]
        for l in range(L):
            e = exg[l]
            for k in range(D // L):
                gbuf_a[base + l, pl.ds(k * L, L)] = (
                    gbuf_a[base + l, pl.ds(k * L, L)] * e)

    pltpu.async_copy(gbuf_a, acc_sp.at[row_r.at[sl, 0]], sem_s, add=True)
    scatter_wait()

    plsc.subcore_barrier()

    pltpu.sync_copy(acc_sp.at[pl.ds(rbase, RPT)],
                    acc_hbm.at[cid, pl.ds(rbase, RPT)])


def _edge(mapped, rowc, colc, exc):
    mesh = plsc.VectorSubcoreMesh(core_axis_name="c", subcore_axis_name="s")
    f = pl.kernel(
        _edge_body,
        out_type=jax.ShapeDtypeStruct((NC, NPAD, D), jnp.float32),
        mesh=mesh,
        scratch_types=[
            pltpu.VMEM((3, 1, K), jnp.int32),
            pltpu.VMEM((3, 1, K), jnp.int32),
            pltpu.VMEM((3, 1, K), jnp.float32),
            pltpu.VMEM((K, D), jnp.float32),
            pltpu.VMEM((K, D), jnp.float32),
            pltpu.VMEM((640,), jnp.float32),
            pltpu.VMEM_SHARED((NPAD, D), jnp.float32),
            pltpu.SemaphoreType.DMA,
            pltpu.SemaphoreType.DMA,
            pltpu.SemaphoreType.DMA,
            pltpu.SemaphoreType.DMA,
            pltpu.SemaphoreType.DMA,
        ],
        compiler_params=_sc_params(),
    )
    return f(mapped, rowc, colc, exc)


def _final_body(acc_ref, den_ref, out_ref):
    acc = acc_ref[0] + acc_ref[1]
    den = den_ref[0, :, :] + den_ref[1, :, :]
    out_ref[...] = acc * (1.0 / jnp.maximum(den, 1e-30))


def _final(acc, den):
    return pl.pallas_call(
        _final_body,
        grid=(N // _BLK,),
        in_specs=[pl.BlockSpec((NC, _BLK, D), lambda i: (0, i, 0)),
                  pl.BlockSpec((NC, _BLK, 1), lambda i: (0, i, 0))],
        out_specs=pl.BlockSpec((_BLK, D), lambda i: (i, 0)),
        out_shape=jax.ShapeDtypeStruct((N, D), jnp.float32),
    )(acc, den)


def kernel(x, edge_index, a_vals, W, M1, M2, gamma, beta, moving_mean,
           moving_var):
    mapped, s1, s2 = _prep(x, W, M1, M2, gamma, beta, moving_mean, moving_var)
    row = edge_index[0]
    col = edge_index[1]
    pad = EPAD - E
    zpad = jnp.zeros((NPAD - N,), jnp.float32)
    s1p = jnp.concatenate([s1.reshape(N), zpad])
    s2p = jnp.concatenate([s2.reshape(N), zpad])
    rowp = jnp.concatenate([row, jnp.full((pad,), N, jnp.int32)])
    colp = jnp.concatenate([col, jnp.zeros((pad,), jnp.int32)])
    ap = jnp.concatenate([a_vals, jnp.zeros((pad,), jnp.float32)])
    ex, den = _exw(s1p, s2p, rowp.reshape(NW, CHUNKS, K),
                   colp.reshape(NW, CHUNKS, K), ap.reshape(NW, CHUNKS, K))
    shape3 = (NW * CHUNKS, 1, K)
    acc = _edge(mapped, rowp.reshape(shape3), colp.reshape(shape3),
                ex.reshape(shape3))
    return _final(acc, den.reshape(NC, NPAD, 1))


# speedup vs baseline: 1.1995x; 1.0176x over previous
import dataclasses
import functools

import jax
import jax.numpy as jnp
from jax import lax
from jax.experimental import pallas as pl
from jax.experimental.pallas import tpu as pltpu
from jax.experimental.pallas import tpu_sc as plsc

N = 10000
D = 128
E = 320000

NC = 2
NS = 16
NW = NC * NS
L = 16

K = 128
NPAD = 10112
EPT = 10112
CHUNKS = EPT // K
EPAD = EPT * NW
CH0 = 125
CH1 = 33
RPT = NPAD // NS

_BLK = 400


def _sc_params():
    cp = pltpu.CompilerParams()
    if "needs_layout_passes" in pltpu.CompilerParams.__dataclass_fields__:
        cp = dataclasses.replace(cp, needs_layout_passes=False)
    return cp


def _prep_body(x_ref, w_ref, m1_ref, m2_ref, g_ref, b_ref, mu_ref, var_ref,
               mapped_ref, s1_ref, s2_ref):
    eps = 1e-3
    scale = g_ref[...] * lax.rsqrt(var_ref[...] + eps)
    xh = (x_ref[...] - mu_ref[...]) * scale + b_ref[...]
    dot = functools.partial(jnp.dot, preferred_element_type=jnp.float32,
                            precision=lax.Precision.HIGHEST)
    mapped_ref[...] = dot(xh, w_ref[...])
    p1 = dot(xh, m1_ref[...])
    s1_ref[...] = jnp.tanh(jnp.sum(p1 * xh, axis=1, keepdims=True))
    p2 = dot(xh, m2_ref[...])
    s2_ref[...] = jnp.tanh(jnp.sum(p2 * xh, axis=1, keepdims=True))


def _prep(x, W, M1, M2, gamma, beta, mu, var):
    full = pl.BlockSpec((D, D), lambda i: (0, 0))
    vec = pl.BlockSpec((1, D), lambda i: (0, 0))
    return pl.pallas_call(
        _prep_body,
        grid=(N // _BLK,),
        in_specs=[pl.BlockSpec((_BLK, D), lambda i: (i, 0)),
                  full, full, full, vec, vec, vec, vec],
        out_specs=[pl.BlockSpec((_BLK, D), lambda i: (i, 0)),
                   pl.BlockSpec((_BLK, 1), lambda i: (i, 0)),
                   pl.BlockSpec((_BLK, 1), lambda i: (i, 0))],
        out_shape=[jax.ShapeDtypeStruct((N, D), jnp.float32),
                   jax.ShapeDtypeStruct((N, 1), jnp.float32),
                   jax.ShapeDtypeStruct((N, 1), jnp.float32)],
    )(x, W, M1, M2, gamma.reshape(1, D), beta.reshape(1, D),
      mu.reshape(1, D), var.reshape(1, D))


def _exw_body(s1_hbm, s2_hbm, row_hbm, col_hbm, a_hbm, ex_hbm, den_hbm,
              s1_v, s2_v, row_v, col_v, a_v, ex_v, zden, den_sp, sem_d):
    cid = lax.axis_index("c")
    sid = lax.axis_index("s")
    wid = cid * NS + sid

    pltpu.sync_copy(s1_hbm, s1_v)
    pltpu.sync_copy(s2_hbm, s2_v)
    pltpu.sync_copy(row_hbm.at[wid], row_v)
    pltpu.sync_copy(col_hbm.at[wid], col_v)
    pltpu.sync_copy(a_hbm.at[wid], a_v)

    zero16 = jnp.zeros((L,), jnp.float32)
    for t in range(640 // L):
        zden[pl.ds(t * L, L)] = zero16
    rbase = sid * RPT
    pltpu.sync_copy(zden.at[pl.ds(0, RPT)], den_sp.at[pl.ds(rbase, RPT)])

    @pl.loop(0, CHUNKS)
    def _chunk(cc):
        for j in range(K // L):
            r16 = row_v[cc, pl.ds(j * L, L)]
            c16 = col_v[cc, pl.ds(j * L, L)]
            av = a_v[cc, pl.ds(j * L, L)]
            g1 = plsc.load_gather(s1_v, [r16])
            g2 = plsc.load_gather(s2_v, [c16])
            ev = av * g1 + av * g2
            ev = jnp.where(ev >= 0.0, ev, 0.2 * ev)
            ex_v[cc, pl.ds(j * L, L)] = jnp.exp(ev)

    pltpu.sync_copy(ex_v, ex_hbm.at[wid])

    plsc.subcore_barrier()

    @pl.loop(0, CHUNKS)
    def _fire(cc):
        pltpu.async_copy(ex_v.at[cc], den_sp.at[row_v.at[cc]], sem_d,
                         add=True)

    @pl.loop(0, CHUNKS)
    def _drain(cc):
        pltpu.make_async_copy(ex_v.at[0], den_sp.at[pl.ds(0, K)],
                              sem_d).wait()

    plsc.subcore_barrier()

    pltpu.sync_copy(den_sp.at[pl.ds(rbase, RPT)], zden.at[pl.ds(0, RPT)])
    pltpu.sync_copy(zden.at[pl.ds(0, RPT)],
                    den_hbm.at[pl.ds(cid * NPAD + rbase, RPT)])


def _exw(s1p, s2p, rowc, colc, ac):
    mesh = plsc.VectorSubcoreMesh(core_axis_name="c", subcore_axis_name="s")
    f = pl.kernel(
        _exw_body,
        out_type=(jax.ShapeDtypeStruct((NW, CHUNKS, K), jnp.float32),
                  jax.ShapeDtypeStruct((NC * NPAD,), jnp.float32)),
        mesh=mesh,
        scratch_types=[
            pltpu.VMEM((NPAD,), jnp.float32),
            pltpu.VMEM((NPAD,), jnp.float32),
            pltpu.VMEM((CHUNKS, K), jnp.int32),
            pltpu.VMEM((CHUNKS, K), jnp.int32),
            pltpu.VMEM((CHUNKS, K), jnp.float32),
            pltpu.VMEM((CHUNKS, K), jnp.float32),
            pltpu.VMEM((640,), jnp.float32),
            pltpu.VMEM_SHARED((NPAD,), jnp.float32),
            pltpu.SemaphoreType.DMA,
        ],
        compiler_params=_sc_params(),
    )
    return f(s1p, s2p, rowc, colc, ac)


def _edge_body(mapped_hbm, row_hbm, col_hbm, ex_hbm,
               acc_hbm,
               row_r, col_r, ex_r, gbuf_a, gbuf_b, zden,
               acc_sp, sem_g, sem_s, sem_row, sem_col, sem_ex):
    cid = lax.axis_index("c")
    sid = lax.axis_index("s")
    nch = jnp.where(cid == 0, CH0, CH1)
    cbase = jnp.where(cid == 0, sid * CH0, NS * CH0 + sid * CH1)

    zero16 = jnp.zeros((L,), jnp.float32)

    @pl.loop(0, K)
    def _zg(i):
        for k in range(D // L):
            gbuf_a[i, pl.ds(k * L, L)] = zero16

    for t in range(640 // L):
        zden[pl.ds(t * L, L)] = zero16

    rbase = sid * RPT
    for t in range(RPT // K):
        pltpu.sync_copy(gbuf_a, acc_sp.at[pl.ds(rbase + t * K, K)])
    rem = RPT - (RPT // K) * K
    if rem:
        pltpu.sync_copy(gbuf_a.at[pl.ds(0, rem)],
                        acc_sp.at[pl.ds(rbase + (RPT // K) * K, rem)])

    plsc.subcore_barrier()

    def idx_start(cc):
        s = lax.rem(cc, 3)
        pltpu.async_copy(row_hbm.at[cbase + cc], row_r.at[s], sem_row)
        pltpu.async_copy(col_hbm.at[cbase + cc], col_r.at[s], sem_col)
        pltpu.async_copy(ex_hbm.at[cbase + cc], ex_r.at[s], sem_ex)

    def idx_wait():
        pltpu.make_async_copy(row_hbm.at[0], row_r.at[0], sem_row).wait()
        pltpu.make_async_copy(col_hbm.at[0], col_r.at[0], sem_col).wait()
        pltpu.make_async_copy(ex_hbm.at[0], ex_r.at[0], sem_ex).wait()

    def gather_start(cc, buf):
        s = lax.rem(cc, 3)
        pltpu.async_copy(mapped_hbm.at[col_r.at[s, 0]], buf, sem_g)

    def gather_wait(buf):
        pltpu.make_async_copy(mapped_hbm.at[pl.ds(0, K)], buf, sem_g).wait()

    def scatter_wait():
        pltpu.make_async_copy(gbuf_a, acc_sp.at[pl.ds(0, K)], sem_s).wait()

    def process(c, buf, other_buf, first_guard, idx_guard):
        s = lax.rem(c, 3)
        idx_wait()
        gather_wait(buf)
        if first_guard:
            @pl.when(c >= 1)
            def _s():
                scatter_wait()
        else:
            scatter_wait()
        gather_start(c + 1, other_buf)
        if idx_guard:
            @pl.when(c + 2 < nch)
            def _i():
                idx_start(c + 2)
        else:
            idx_start(c + 2)

        @pl.loop(0, K // L)
        def _scale(grp):
            base = pl.multiple_of(grp * L, L)
            exg = ex_r[s, 0, pl.ds(base, L)]
            for l in range(L):
                i = base + l
                e = exg[l]
                for k in range(D // L):
                    buf[i, pl.ds(k * L, L)] = buf[i, pl.ds(k * L, L)] * e

        pltpu.async_copy(buf, acc_sp.at[row_r.at[s, 0]], sem_s, add=True)

    idx_start(0)
    idx_start(1)
    idx_wait()
    gather_start(0, gbuf_a)

    @pl.loop(0, (nch - 1) // 2)
    def _pipe(p):
        a = pl.multiple_of(2 * p, 2)
        process(a, gbuf_a, gbuf_b, True, False)
        process(a + 1, gbuf_b, gbuf_a, False, True)

    cl = nch - 1
    sl = lax.rem(cl, 3)
    gather_wait(gbuf_a)
    scatter_wait()

    @pl.loop(0, K // L)
    def _scale_last(grp):
        base = pl.multiple_of(grp * L, L)
        exg = ex_r[sl, 0, pl.ds(base, L)]
        for l in range(L):
            e = exg[l]
            for k in range(D // L):
                gbuf_a[base + l, pl.ds(k * L, L)] = (
                    gbuf_a[base + l, pl.ds(k * L, L)] * e)

    pltpu.async_copy(gbuf_a, acc_sp.at[row_r.at[sl, 0]], sem_s, add=True)
    scatter_wait()

    plsc.subcore_barrier()

    pltpu.sync_copy(acc_sp.at[pl.ds(rbase, RPT)],
                    acc_hbm.at[cid, pl.ds(rbase, RPT)])


def _edge(mapped, rowc, colc, exc):
    mesh = plsc.VectorSubcoreMesh(core_axis_name="c", subcore_axis_name="s")
    f = pl.kernel(
        _edge_body,
        out_type=jax.ShapeDtypeStruct((NC, NPAD, D), jnp.float32),
        mesh=mesh,
        scratch_types=[
            pltpu.VMEM((3, 1, K), jnp.int32),
            pltpu.VMEM((3, 1, K), jnp.int32),
            pltpu.VMEM((3, 1, K), jnp.float32),
            pltpu.VMEM((K, D), jnp.float32),
            pltpu.VMEM((K, D), jnp.float32),
            pltpu.VMEM((640,), jnp.float32),
            pltpu.VMEM_SHARED((NPAD, D), jnp.float32),
            pltpu.SemaphoreType.DMA,
            pltpu.SemaphoreType.DMA,
            pltpu.SemaphoreType.DMA,
            pltpu.SemaphoreType.DMA,
            pltpu.SemaphoreType.DMA,
        ],
        compiler_params=_sc_params(),
    )
    return f(mapped, rowc, colc, exc)


def _final_body(acc_ref, den_ref, out_ref):
    acc = acc_ref[0] + acc_ref[1]
    den = den_ref[0, :, :] + den_ref[1, :, :]
    out_ref[...] = acc * (1.0 / jnp.maximum(den, 1e-30))


def _final(acc, den):
    return pl.pallas_call(
        _final_body,
        grid=(N // _BLK,),
        in_specs=[pl.BlockSpec((NC, _BLK, D), lambda i: (0, i, 0)),
                  pl.BlockSpec((NC, _BLK, 1), lambda i: (0, i, 0))],
        out_specs=pl.BlockSpec((_BLK, D), lambda i: (i, 0)),
        out_shape=jax.ShapeDtypeStruct((N, D), jnp.float32),
    )(acc, den)


def kernel(x, edge_index, a_vals, W, M1, M2, gamma, beta, moving_mean,
           moving_var):
    mapped, s1, s2 = _prep(x, W, M1, M2, gamma, beta, moving_mean, moving_var)
    row = edge_index[0]
    col = edge_index[1]
    pad = EPAD - E
    zpad = jnp.zeros((NPAD - N,), jnp.float32)
    s1p = jnp.concatenate([s1.reshape(N), zpad])
    s2p = jnp.concatenate([s2.reshape(N), zpad])
    rowp = jnp.concatenate([row, jnp.full((pad,), N, jnp.int32)])
    colp = jnp.concatenate([col, jnp.zeros((pad,), jnp.int32)])
    ap = jnp.concatenate([a_vals, jnp.zeros((pad,), jnp.float32)])
    ex, den = _exw(s1p, s2p, rowp.reshape(NW, CHUNKS, K),
                   colp.reshape(NW, CHUNKS, K), ap.reshape(NW, CHUNKS, K))
    shape3 = (NW * CHUNKS, 1, K)
    acc = _edge(mapped, rowp.reshape(shape3), colp.reshape(shape3),
                ex.reshape(shape3))
    return _final(acc, den.reshape(NC, NPAD, 1))
